# Initial kernel scaffold; baseline (speedup 1.0000x reference)
import jax
import jax.numpy as jnp
from jax.experimental import pallas as pl


def kernel(x, edge_index, W1, b1, W2, b2):
    raise NotImplementedError("write your pallas kernel here")


# speedup vs baseline: 7.7713x; 7.7713x over previous
import functools

import jax
import jax.numpy as jnp
from jax import lax
from jax.experimental import pallas as pl
from jax.experimental.pallas import tpu as pltpu
from jax.experimental.pallas import tpu_sc as plsc

NC = 2
NS = 16
CHUNK = 128


def _sc_mesh():
    return plsc.VectorSubcoreMesh(
        core_axis_name="c", subcore_axis_name="s", num_cores=NC, num_subcores=NS
    )


def _sc_degree_call(dst_p, n, npad, e_pad):
    per_w = e_pad // (NC * NS)
    nchunks = per_w // CHUNK
    drain = n // 10

    @functools.partial(
        pl.kernel,
        mesh=_sc_mesh(),
        out_type=jax.ShapeDtypeStruct((2 * n, 128), jnp.float32),
        scratch_types=[
            pltpu.VMEM_SHARED((npad, 128), jnp.float32),
            pltpu.VMEM((CHUNK, 128), jnp.float32),
            pltpu.VMEM((1, CHUNK), jnp.int32),
        ],
    )
    def deg_k(dst_hbm, ones_hbm, zeros_hbm, out_hbm, acc, ones_v, idx_v):
        c = lax.axis_index("c")
        s = lax.axis_index("s")
        w = s * NC + c
        rows_per = npad // NS
        pltpu.sync_copy(zeros_hbm, acc.at[pl.ds(s * rows_per, rows_per)])
        pltpu.sync_copy(ones_hbm, ones_v)
        plsc.subcore_barrier()
        base = w * per_w

        def body(k, carry):
            b0 = base + k * CHUNK
            pltpu.sync_copy(dst_hbm.at[pl.ds(b0, CHUNK)], idx_v.at[0])
            pltpu.sync_copy(ones_v, acc.at[idx_v.at[0]], add=True)
            return carry

        lax.fori_loop(0, nchunks, body, 0)
        plsc.subcore_barrier()

        @pl.when(s < 10)
        def _():
            pltpu.sync_copy(
                acc.at[pl.ds(s * drain, drain)],
                out_hbm.at[pl.ds(c * n + s * drain, drain)],
            )

    ones = jnp.ones((CHUNK, 128), jnp.float32)
    zeros = jnp.zeros((npad // NS, 128), jnp.float32)
    return deg_k(dst_p, ones, zeros)


def _sc_conv_call(table, src_p, dst_p, n, npad, e_pad, split_cols):
    if split_cols:
        per = e_pad // NS
    else:
        per = e_pad // (NC * NS)
    nchunks = per // CHUNK
    drain = n // 10

    @functools.partial(
        pl.kernel,
        mesh=_sc_mesh(),
        out_type=jax.ShapeDtypeStruct((2 * n, 128), jnp.float32),
        scratch_types=[
            pltpu.VMEM_SHARED((npad, 128), jnp.float32),
            pltpu.VMEM((1, CHUNK), jnp.int32),
            pltpu.VMEM((1, CHUNK), jnp.int32),
            pltpu.VMEM((CHUNK, 128), jnp.float32),
            pltpu.SemaphoreType.DMA,
        ],
    )
    def conv_k(table_hbm, src_hbm, dst_hbm, zeros_hbm, out_hbm,
               acc, sidx, didx, rows_v, sem):
        c = lax.axis_index("c")
        s = lax.axis_index("s")
        rows_per = npad // NS
        pltpu.sync_copy(zeros_hbm, acc.at[pl.ds(s * rows_per, rows_per)])
        plsc.subcore_barrier()
        if split_cols:
            base = s * per
            roff = c * n
        else:
            base = (s * NC + c) * per
            roff = None

        def body(k, carry):
            b0 = base + k * CHUNK
            pltpu.sync_copy(src_hbm.at[pl.ds(b0, CHUNK)], sidx.at[0])
            pltpu.sync_copy(dst_hbm.at[pl.ds(b0, CHUNK)], didx.at[0])
            if roff is not None:
                for l in range(CHUNK // 16):
                    sidx[0, pl.ds(l * 16, 16)] = (
                        sidx[0, pl.ds(l * 16, 16)] + roff
                    )
            pltpu.async_copy(table_hbm.at[sidx.at[0]], rows_v, sem).wait()
            pltpu.sync_copy(rows_v, acc.at[didx.at[0]], add=True)
            return carry

        lax.fori_loop(0, nchunks, body, 0)
        plsc.subcore_barrier()

        @pl.when(s < 10)
        def _():
            pltpu.sync_copy(
                acc.at[pl.ds(s * drain, drain)],
                out_hbm.at[pl.ds(c * n + s * drain, drain)],
            )

    zeros = jnp.zeros((npad // NS, 128), jnp.float32)
    return conv_k(table, src_p, dst_p, zeros)


def _dinv_from(d0, d1):
    return lax.rsqrt(d0[:, 0:1] + d1[:, 0:1] + 1.0)


def _tc_mm1_call(x, W1, degp, n, bm):
    g = n // bm

    def body(x_ref, w_ref, d0, d1, o_ref):
        dinv = _dinv_from(d0[...], d1[...])
        m = jnp.dot(x_ref[...], w_ref[...],
                    preferred_element_type=jnp.float32) * dinv
        o_ref[0, :, :] = m[:, :128]
        o_ref[1, :, :] = m[:, 128:]

    return pl.pallas_call(
        body,
        grid=(g,),
        in_specs=[
            pl.BlockSpec((bm, 256), lambda i: (i, 0)),
            pl.BlockSpec((256, 256), lambda i: (0, 0)),
            pl.BlockSpec((bm, 128), lambda i: (i, 0)),
            pl.BlockSpec((bm, 128), lambda i: (i + g, 0)),
        ],
        out_specs=pl.BlockSpec((2, bm, 128), lambda i: (0, i, 0)),
        out_shape=jax.ShapeDtypeStruct((2, n, 128), jnp.float32),
    )(x, W1, degp, degp)


def _tc_mm2_call(s1, g1, degp, W2, b1, n, bm):
    g = n // bm

    def body(s1_ref, g1_ref, d0, d1, w2_ref, b1_ref, o_ref):
        dinv = _dinv_from(d0[...], d1[...])
        h0 = jax.nn.relu(dinv * (s1_ref[0, :, :] + g1_ref[0, :, :])
                         + b1_ref[0:1, 0:128])
        h1 = jax.nn.relu(dinv * (s1_ref[1, :, :] + g1_ref[1, :, :])
                         + b1_ref[0:1, 128:256])
        m = jnp.dot(h0, w2_ref[0, :, :], preferred_element_type=jnp.float32)
        m = m + jnp.dot(h1, w2_ref[1, :, :],
                        preferred_element_type=jnp.float32)
        o_ref[...] = dinv * m

    return pl.pallas_call(
        body,
        grid=(g,),
        in_specs=[
            pl.BlockSpec((2, bm, 128), lambda i: (0, i, 0)),
            pl.BlockSpec((2, bm, 128), lambda i: (0, i, 0)),
            pl.BlockSpec((bm, 128), lambda i: (i, 0)),
            pl.BlockSpec((bm, 128), lambda i: (i + g, 0)),
            pl.BlockSpec((2, 128, 128), lambda i: (0, 0, 0)),
            pl.BlockSpec((1, 256), lambda i: (0, 0)),
        ],
        out_specs=pl.BlockSpec((bm, 128), lambda i: (i, 0)),
        out_shape=jax.ShapeDtypeStruct((n, 128), jnp.float32),
    )(s1, g1, degp, degp, W2.reshape(2, 128, 128), b1.reshape(1, 256))


def _tc_zfin_call(s2, g2, degp, b2, n, bm):
    g = n // bm

    def body(s2_ref, g2_ref, d0, d1, b2_ref, o_ref):
        dinv = _dinv_from(d0[...], d1[...])
        o_ref[...] = (dinv * (s2_ref[0, :, :] + s2_ref[1, :, :] + g2_ref[...])
                      + b2_ref[0:1, :])

    return pl.pallas_call(
        body,
        grid=(g,),
        in_specs=[
            pl.BlockSpec((2, bm, 128), lambda i: (0, i, 0)),
            pl.BlockSpec((bm, 128), lambda i: (i, 0)),
            pl.BlockSpec((bm, 128), lambda i: (i, 0)),
            pl.BlockSpec((bm, 128), lambda i: (i + g, 0)),
            pl.BlockSpec((1, 128), lambda i: (0, 0)),
        ],
        out_specs=pl.BlockSpec((bm, 128), lambda i: (i, 0)),
        out_shape=jax.ShapeDtypeStruct((n, 128), jnp.float32),
    )(s2, g2, degp, degp, b2.reshape(1, 128))


def _tc_decoder_call(z, n, bm):
    g = n // bm

    def body(zi_ref, zall_ref, o_ref):
        acc = lax.dot_general(
            zi_ref[...], zall_ref[...],
            (((1,), (1,)), ((), ())),
            preferred_element_type=jnp.float32,
        )
        o_ref[...] = jax.nn.sigmoid(acc)

    return pl.pallas_call(
        body,
        grid=(g,),
        in_specs=[
            pl.BlockSpec((bm, 128), lambda i: (i, 0)),
            pl.BlockSpec((n, 128), lambda i: (0, 0)),
        ],
        out_specs=pl.BlockSpec((bm, n), lambda i: (i, 0)),
        out_shape=jax.ShapeDtypeStruct((n, n), jnp.float32),
    )(z, z)


def kernel(x, edge_index, W1, b1, W2, b2):
    n = x.shape[0]
    e = edge_index.shape[1]
    e_pad = -(-e // 4096) * 4096
    npad = (-(-(n + 16) // 128)) * 128

    src = edge_index[0]
    dst = edge_index[1]
    pad = e_pad - e
    if pad:
        src_p = jnp.concatenate([src, jnp.zeros((pad,), jnp.int32)])
        dst_p = jnp.concatenate([dst, jnp.full((pad,), n, jnp.int32)])
    else:
        src_p, dst_p = src, dst

    degp = _sc_degree_call(dst_p, n, npad, e_pad)
    g1 = _tc_mm1_call(x, W1, degp, n, bm=1000)
    s1 = _sc_conv_call(g1.reshape(2 * n, 128), src_p, dst_p,
                       n, npad, e_pad, split_cols=True)
    g2 = _tc_mm2_call(s1.reshape(2, n, 128), g1, degp, W2, b1,
                      n, bm=1000)
    s2 = _sc_conv_call(g2, src_p, dst_p, n, npad, e_pad,
                       split_cols=False)
    z = _tc_zfin_call(s2.reshape(2, n, 128), g2, degp, b2, n, bm=1000)
    adj = _tc_decoder_call(z, n, bm=400)
    return adj, z
